# baseline (speedup 1.0000x reference)
import jax
import jax.numpy as jnp
from jax import lax
from jax.experimental import pallas as pl
from jax.experimental.pallas import tpu as pltpu
from jax.experimental.pallas import tpu_sc as plsc

N = 10000
D = 256
HALF = 128
E = 160000
NC = 2
NS = 16
L = 16
NP = 10240
RT = NP // NS
RC = RT // 128
MW = 128
MC = RT // MW
EPT = 10240
CH = 80
C = 128
W = 16
NWIN = CH // W


def _sc_body(xp, row3, col3, u1f, u2f, dvf, gf,
             rowv, colv, buf0, hist, redbuf, dinv, shist, acc):
    c = lax.axis_index("c")
    s = lax.axis_index("s")
    goff = c * NP
    coff = c * HALF

    ones = jnp.full((L,), 1.0, jnp.float32)
    zeros = jnp.zeros((L,), jnp.float32)

    def zero_buf0():
        def zb(r, _):
            for q in range(HALF // L):
                buf0[r, pl.ds(q * L, L)] = zeros
            return 0
        lax.fori_loop(0, C, zb, 0)

    def z1(i, _):
        hist[pl.ds(i * L, L)] = zeros
        return 0
    lax.fori_loop(0, NP // L, z1, 0)

    for w in range(NWIN):
        pltpu.sync_copy(col3.at[s, pl.ds(w * W, W)], colv)

        def p0(j, _):
            for q in range(C // L):
                cv = colv[j, pl.ds(q * L, L)]
                plsc.addupdate_scatter(hist, [cv], ones)
            return 0
        lax.fori_loop(0, W, p0, 0)

    pltpu.sync_copy(hist, shist.at[s])
    plsc.subcore_barrier()

    for t in range(MC):
        pltpu.sync_copy(shist.at[:, pl.ds(s * RT + t * MW, MW)], redbuf)

        def p1(i, _, _t=t):
            d = redbuf[0, pl.ds(i * L, L)]
            for u in range(1, NS):
                d = d + redbuf[u, pl.ds(i * L, L)]
            dsafe = jnp.maximum(d, 1.0)
            bi = plsc.bitcast(dsafe, jnp.int32)
            bi = jnp.int32(0x5F3759DF) - lax.shift_right_logical(bi, 1)
            y = plsc.bitcast(bi, jnp.float32)
            for _ in range(3):
                y = y * (1.5 - 0.5 * dsafe * y * y)
            dinv[pl.ds(_t * MW + i * L, L)] = jnp.where(d >= 0.5, y, 0.0)
            return 0
        lax.fori_loop(0, MW // L, p1, 0)

    @pl.when(c == 0)
    def _():
        pltpu.sync_copy(dinv, dvf.at[pl.ds(s * RT, RT)])

    zero_buf0()
    for t in range(RC):
        pltpu.sync_copy(buf0, acc.at[pl.ds(s * RT + t * 128, 128)])

    def scale_buf0(t, squared):
        def sc1(r, _, _t=t):
            dv = plsc.load_gather(dinv, [jnp.full((L,), _t * 128 + r, jnp.int32)])
            if squared:
                dv = dv * dv
            for q in range(HALF // L):
                buf0[r, pl.ds(q * L, L)] = buf0[r, pl.ds(q * L, L)] * dv
            return 0
        lax.fori_loop(0, 128, sc1, 0)

    for t in range(RC):
        r0 = s * RT + t * 128
        pltpu.sync_copy(xp.at[pl.ds(r0, 128), pl.ds(coff, HALF)], buf0)
        scale_buf0(t, False)
        pltpu.sync_copy(buf0, gf.at[pl.ds(goff + r0, 128)])
    plsc.subcore_barrier()

    def propagate():
        for w in range(NWIN):
            pltpu.sync_copy(row3.at[s, pl.ds(w * W, W)], rowv)
            pltpu.sync_copy(col3.at[s, pl.ds(w * W, W)], colv)

            def adj(j, _):
                for q in range(C // L):
                    rv = rowv[j, pl.ds(q * L, L)]
                    rowv[j, pl.ds(q * L, L)] = rv + goff
                return 0
            lax.fori_loop(0, W, adj, 0)

            def p2(j, _):
                pltpu.sync_copy(gf.at[rowv.at[j]], buf0)
                pltpu.sync_copy(buf0, acc.at[colv.at[j]], add=True)
                return 0
            lax.fori_loop(0, W, p2, 0)
        plsc.subcore_barrier()

    propagate()
    for t in range(RC):
        r0 = s * RT + t * 128
        pltpu.sync_copy(acc.at[pl.ds(r0, 128)], u1f.at[pl.ds(goff + r0, 128)])
        pltpu.sync_copy(acc.at[pl.ds(r0, 128)], buf0)
        scale_buf0(t, True)
        pltpu.sync_copy(buf0, gf.at[pl.ds(goff + r0, 128)])
        zero_buf0()
        pltpu.sync_copy(buf0, acc.at[pl.ds(r0, 128)])
    plsc.subcore_barrier()

    propagate()
    for t in range(RC):
        r0 = s * RT + t * 128
        pltpu.sync_copy(acc.at[pl.ds(r0, 128)], u2f.at[pl.ds(goff + r0, 128)])


def _tc0_body(x_ref, w0, b_ref, o_ref):
    o_ref[...] = jnp.dot(x_ref[...], w0[...],
                         preferred_element_type=jnp.float32) + b_ref[...]


def _tc_body(p_ref, dv_ref, u1a, u1b, u2a, u2b, w1, w2, o_ref):
    dv = dv_ref[...]
    acc = p_ref[...]
    acc += jnp.dot(u1a[...] * dv, w1[:HALF, :],
                   preferred_element_type=jnp.float32)
    acc += jnp.dot(u1b[...] * dv, w1[HALF:, :],
                   preferred_element_type=jnp.float32)
    acc += jnp.dot(u2a[...] * dv, w2[:HALF, :],
                   preferred_element_type=jnp.float32)
    acc += jnp.dot(u2b[...] * dv, w2[HALF:, :],
                   preferred_element_type=jnp.float32)
    o_ref[...] = acc


def kernel(x, edge_index, Ws, b):
    xp = jnp.pad(x, ((0, NP - N), (0, 0)))
    pad = NS * EPT - E
    rowp = jnp.concatenate([edge_index[0], jnp.zeros((pad,), jnp.int32)])
    colp = jnp.concatenate([edge_index[1], jnp.full((pad,), N, jnp.int32)])
    row3 = rowp.reshape(NS, CH, C)
    col3 = colp.reshape(NS, CH, C)

    f32 = jnp.float32
    sc = pl.kernel(
        _sc_body,
        out_type=[
            jax.ShapeDtypeStruct((NC * NP, HALF), f32),
            jax.ShapeDtypeStruct((NC * NP, HALF), f32),
            jax.ShapeDtypeStruct((NP,), f32),
            jax.ShapeDtypeStruct((NC * NP, HALF), f32),
        ],
        mesh=plsc.VectorSubcoreMesh(core_axis_name="c", subcore_axis_name="s"),
        compiler_params=pltpu.CompilerParams(needs_layout_passes=False),
        scratch_types=[
            pltpu.VMEM((W, C), jnp.int32),
            pltpu.VMEM((W, C), jnp.int32),
            pltpu.VMEM((C, HALF), f32),
            pltpu.VMEM((NP,), f32),
            pltpu.VMEM((NS, MW), f32),
            pltpu.VMEM((RT,), f32),
            pltpu.VMEM_SHARED((NS, NP), f32),
            pltpu.VMEM_SHARED((NP, HALF), f32),
        ],
    )
    grid = (NP // 1024,)
    part = pl.pallas_call(
        _tc0_body,
        grid=grid,
        in_specs=[
            pl.BlockSpec((1024, D), lambda i: (i, 0)),
            pl.BlockSpec((D, D), lambda i: (0, 0)),
            pl.BlockSpec((1, D), lambda i: (0, 0)),
        ],
        out_specs=pl.BlockSpec((1024, D), lambda i: (i, 0)),
        out_shape=jax.ShapeDtypeStruct((NP, D), f32),
    )(xp, Ws[0], b.reshape(1, D))

    u1f, u2f, dvf, _ = sc(xp, row3, col3)
    dvr = jnp.broadcast_to(dvf.reshape(NP, 1), (NP, HALF))

    out = pl.pallas_call(
        _tc_body,
        grid=grid,
        in_specs=[
            pl.BlockSpec((1024, D), lambda i: (i, 0)),
            pl.BlockSpec((1024, HALF), lambda i: (i, 0)),
            pl.BlockSpec((1024, HALF), lambda i: (i, 0)),
            pl.BlockSpec((1024, HALF), lambda i: (i + NP // 1024, 0)),
            pl.BlockSpec((1024, HALF), lambda i: (i, 0)),
            pl.BlockSpec((1024, HALF), lambda i: (i + NP // 1024, 0)),
            pl.BlockSpec((D, D), lambda i: (0, 0)),
            pl.BlockSpec((D, D), lambda i: (0, 0)),
        ],
        out_specs=pl.BlockSpec((1024, D), lambda i: (i, 0)),
        out_shape=jax.ShapeDtypeStruct((N, D), f32),
    )(part, dvr, u1f, u1f, u2f, u2f, Ws[1], Ws[2])
    return out


# speedup vs baseline: 1.0060x; 1.0060x over previous
import jax
import jax.numpy as jnp
from jax import lax
from jax.experimental import pallas as pl
from jax.experimental.pallas import tpu as pltpu
from jax.experimental.pallas import tpu_sc as plsc

N = 10000
D = 256
HALF = 128
E = 160000
NC = 2
NS = 16
L = 16
NP = 10240
RT = NP // NS
RC = RT // 128
MW = 128
MC = RT // MW
EPT = 10240
CH = 80
C = 128
W = 16
NWIN = CH // W


def _sc_body(xp, row3, col3, u1f, u2f, dvf, gf,
             rowv, colv, buf0, hist, redbuf, dinv, shist, acc):
    c = lax.axis_index("c")
    s = lax.axis_index("s")
    goff = c * NP
    coff = c * HALF

    ones = jnp.full((L,), 1.0, jnp.float32)
    zeros = jnp.zeros((L,), jnp.float32)

    def zero_buf0():
        def zb(r, _):
            for q in range(HALF // L):
                buf0[r, pl.ds(q * L, L)] = zeros
            return 0
        lax.fori_loop(0, C, zb, 0)

    def z1(i, _):
        hist[pl.ds(i * L, L)] = zeros
        return 0
    lax.fori_loop(0, NP // L, z1, 0)

    for w in range(NWIN):
        pltpu.sync_copy(col3.at[s, pl.ds(w * W, W)], colv)

        def p0(j, _):
            for q in range(C // L):
                cv = colv[j, pl.ds(q * L, L)]
                plsc.addupdate_scatter(hist, [cv], ones)
            return 0
        lax.fori_loop(0, W, p0, 0)

    pltpu.sync_copy(hist, shist.at[s])
    plsc.subcore_barrier()

    for t in range(MC):
        pltpu.sync_copy(shist.at[:, pl.ds(s * RT + t * MW, MW)], redbuf)

        def p1(i, _, _t=t):
            d = redbuf[0, pl.ds(i * L, L)]
            for u in range(1, NS):
                d = d + redbuf[u, pl.ds(i * L, L)]
            dsafe = jnp.maximum(d, 1.0)
            bi = plsc.bitcast(dsafe, jnp.int32)
            bi = jnp.int32(0x5F3759DF) - lax.shift_right_logical(bi, 1)
            y = plsc.bitcast(bi, jnp.float32)
            for _ in range(3):
                y = y * (1.5 - 0.5 * dsafe * y * y)
            dinv[pl.ds(_t * MW + i * L, L)] = jnp.where(d >= 0.5, y, 0.0)
            return 0
        lax.fori_loop(0, MW // L, p1, 0)

    @pl.when(c == 0)
    def _():
        pltpu.sync_copy(dinv, dvf.at[pl.ds(s * RT, RT)])

    zero_buf0()
    for t in range(RC):
        pltpu.sync_copy(buf0, acc.at[pl.ds(s * RT + t * 128, 128)])

    def scale_buf0(t, squared):
        def sc1(r, _, _t=t):
            dv = plsc.load_gather(dinv, [jnp.full((L,), _t * 128 + r, jnp.int32)])
            if squared:
                dv = dv * dv
            for q in range(HALF // L):
                buf0[r, pl.ds(q * L, L)] = buf0[r, pl.ds(q * L, L)] * dv
            return 0
        lax.fori_loop(0, 128, sc1, 0)

    for t in range(RC):
        r0 = s * RT + t * 128
        pltpu.sync_copy(xp.at[pl.ds(r0, 128), pl.ds(coff, HALF)], buf0)
        scale_buf0(t, False)
        pltpu.sync_copy(buf0, gf.at[pl.ds(goff + r0, 128)])
    plsc.subcore_barrier()

    def propagate():
        for w in range(NWIN):
            pltpu.sync_copy(row3.at[s, pl.ds(w * W, W)], rowv)
            pltpu.sync_copy(col3.at[s, pl.ds(w * W, W)], colv)

            def adj(j, _):
                for q in range(C // L):
                    rv = rowv[j, pl.ds(q * L, L)]
                    rowv[j, pl.ds(q * L, L)] = rv + goff
                return 0
            lax.fori_loop(0, W, adj, 0)

            def p2(j, _):
                pltpu.sync_copy(gf.at[rowv.at[j]], buf0)
                pltpu.sync_copy(buf0, acc.at[colv.at[j]], add=True)
                return 0
            lax.fori_loop(0, W, p2, 0)
        plsc.subcore_barrier()

    propagate()
    for t in range(RC):
        r0 = s * RT + t * 128
        pltpu.sync_copy(acc.at[pl.ds(r0, 128)], u1f.at[pl.ds(goff + r0, 128)])
        pltpu.sync_copy(acc.at[pl.ds(r0, 128)], buf0)
        scale_buf0(t, True)
        pltpu.sync_copy(buf0, gf.at[pl.ds(goff + r0, 128)])
        zero_buf0()
        pltpu.sync_copy(buf0, acc.at[pl.ds(r0, 128)])
    plsc.subcore_barrier()

    propagate()
    for t in range(RC):
        r0 = s * RT + t * 128
        pltpu.sync_copy(acc.at[pl.ds(r0, 128)], u2f.at[pl.ds(goff + r0, 128)])


def _tc0_body(x_ref, w0, b_ref, o_ref):
    o_ref[...] = jnp.dot(x_ref[...], w0[...],
                         preferred_element_type=jnp.float32) + b_ref[...]


def _tc_body(p_ref, dv_ref, u1a, u1b, u2a, u2b, w1, w2, o_ref):
    dv = dv_ref[...]
    acc = p_ref[...]
    acc += jnp.dot(u1a[...] * dv, w1[:HALF, :],
                   preferred_element_type=jnp.float32)
    acc += jnp.dot(u1b[...] * dv, w1[HALF:, :],
                   preferred_element_type=jnp.float32)
    acc += jnp.dot(u2a[...] * dv, w2[:HALF, :],
                   preferred_element_type=jnp.float32)
    acc += jnp.dot(u2b[...] * dv, w2[HALF:, :],
                   preferred_element_type=jnp.float32)
    o_ref[...] = acc


def kernel(x, edge_index, Ws, b):
    xp = jnp.pad(x, ((0, NP - N), (0, 0)))
    pad = NS * EPT - E
    rowp = jnp.concatenate([edge_index[0], jnp.zeros((pad,), jnp.int32)])
    padcol = N + (jnp.arange(pad, dtype=jnp.int32) % (NP - N))
    colp = jnp.concatenate([edge_index[1], padcol])
    row3 = rowp.reshape(NS, CH, C)
    col3 = colp.reshape(NS, CH, C)

    f32 = jnp.float32
    sc = pl.kernel(
        _sc_body,
        out_type=[
            jax.ShapeDtypeStruct((NC * NP, HALF), f32),
            jax.ShapeDtypeStruct((NC * NP, HALF), f32),
            jax.ShapeDtypeStruct((NP,), f32),
            jax.ShapeDtypeStruct((NC * NP, HALF), f32),
        ],
        mesh=plsc.VectorSubcoreMesh(core_axis_name="c", subcore_axis_name="s"),
        compiler_params=pltpu.CompilerParams(needs_layout_passes=False),
        scratch_types=[
            pltpu.VMEM((W, C), jnp.int32),
            pltpu.VMEM((W, C), jnp.int32),
            pltpu.VMEM((C, HALF), f32),
            pltpu.VMEM((NP,), f32),
            pltpu.VMEM((NS, MW), f32),
            pltpu.VMEM((RT,), f32),
            pltpu.VMEM_SHARED((NS, NP), f32),
            pltpu.VMEM_SHARED((NP, HALF), f32),
        ],
    )
    grid = (NP // 1024,)
    part = pl.pallas_call(
        _tc0_body,
        grid=grid,
        in_specs=[
            pl.BlockSpec((1024, D), lambda i: (i, 0)),
            pl.BlockSpec((D, D), lambda i: (0, 0)),
            pl.BlockSpec((1, D), lambda i: (0, 0)),
        ],
        out_specs=pl.BlockSpec((1024, D), lambda i: (i, 0)),
        out_shape=jax.ShapeDtypeStruct((NP, D), f32),
    )(xp, Ws[0], b.reshape(1, D))

    u1f, u2f, dvf, _ = sc(xp, row3, col3)
    dvr = jnp.broadcast_to(dvf.reshape(NP, 1), (NP, HALF))

    out = pl.pallas_call(
        _tc_body,
        grid=grid,
        in_specs=[
            pl.BlockSpec((1024, D), lambda i: (i, 0)),
            pl.BlockSpec((1024, HALF), lambda i: (i, 0)),
            pl.BlockSpec((1024, HALF), lambda i: (i, 0)),
            pl.BlockSpec((1024, HALF), lambda i: (i + NP // 1024, 0)),
            pl.BlockSpec((1024, HALF), lambda i: (i, 0)),
            pl.BlockSpec((1024, HALF), lambda i: (i + NP // 1024, 0)),
            pl.BlockSpec((D, D), lambda i: (0, 0)),
            pl.BlockSpec((D, D), lambda i: (0, 0)),
        ],
        out_specs=pl.BlockSpec((1024, D), lambda i: (i, 0)),
        out_shape=jax.ShapeDtypeStruct((N, D), f32),
    )(part, dvr, u1f, u1f, u2f, u2f, Ws[1], Ws[2])
    return out
